# baseline (speedup 1.0000x reference)
import dataclasses
import functools

import jax
import jax.numpy as jnp
from jax import lax
from jax.experimental import pallas as pl
from jax.experimental.pallas import tpu as pltpu
from jax.experimental.pallas import tpu_sc as plsc

N = 2048
D = 768
E = 64
K = 2
G = 8
EPG = E // G
TG = 4
F = 512
SF = 512
SCALE = 2.5

BT = 256
BM = 128
NBMAX = E + N * K // BM - 32
SLOTS = NBMAX * BM

_NEG = -1e30


def _router_body(scores_ref, ebias_ref, idx_ref, w_ref, r_ref, po_ref,
                 te_ref, tb_ref, tv_ref, carry_ref):
    pid = pl.program_id(0)

    @pl.when(pid == 0)
    def _():
        carry_ref[...] = jnp.zeros((1, E), jnp.float32)

    s = scores_ref[...]
    sr = s + ebias_ref[...]
    eid = lax.broadcasted_iota(jnp.int32, (BT, E), 1)

    gs_cols = []
    for g in range(G):
        gmask = (eid >= g * EPG) & (eid < (g + 1) * EPG)
        v = jnp.where(gmask, sr, _NEG)
        m1 = jnp.max(v, axis=-1, keepdims=True)
        i1 = jnp.min(jnp.where(v == m1, eid, E), axis=-1, keepdims=True)
        m2 = jnp.max(jnp.where(eid == i1, _NEG, v), axis=-1, keepdims=True)
        gs_cols.append(m1 + m2)
    gs = jnp.concatenate(gs_cols, axis=1)

    gio = lax.broadcasted_iota(jnp.int32, (BT, G), 1)
    sel = jnp.zeros((BT, G), jnp.bool_)
    for _ in range(TG):
        mg = jnp.max(gs, axis=-1, keepdims=True)
        ig = jnp.min(jnp.where(gs == mg, gio, G), axis=-1, keepdims=True)
        sel = sel | (gio == ig)
        gs = jnp.where(gio == ig, _NEG, gs)

    msk = jnp.full((BT, E), _NEG, jnp.float32)
    for g in range(G):
        gmask = (eid >= g * EPG) & (eid < (g + 1) * EPG)
        msk = jnp.where(gmask & sel[:, g:g + 1], sr, msk)
    picks = []
    for _ in range(K):
        mk = jnp.max(msk, axis=-1, keepdims=True)
        ik = jnp.min(jnp.where(msk == mk, eid, E), axis=-1, keepdims=True)
        wk = jnp.sum(jnp.where(eid == ik, s, 0.0), axis=-1, keepdims=True)
        msk = jnp.where(eid == ik, _NEG, msk)
        picks.append((ik, wk))
    (i0, w0), (i1, w1) = picks
    denom = (w0 + w1) + 1e-20
    w0n = (w0 / denom) * SCALE
    w1n = (w1 / denom) * SCALE

    onehots = (eid == i0).astype(jnp.float32) + (eid == i1).astype(jnp.float32)
    rio = lax.broadcasted_iota(jnp.int32, (BT, BT), 0)
    cio = lax.broadcasted_iota(jnp.int32, (BT, BT), 1)
    ltri = (cio <= rio).astype(jnp.float32)
    cs = jnp.dot(ltri, onehots, preferred_element_type=jnp.float32)
    tot = cs + carry_ref[...]
    r0 = jnp.sum(jnp.where(eid == i0, tot, 0.0), axis=-1, keepdims=True) - 1.0
    r1 = jnp.sum(jnp.where(eid == i1, tot, 0.0), axis=-1, keepdims=True) - 1.0
    newc = carry_ref[...] + cs[BT - 1:BT, :]
    carry_ref[...] = newc

    idx_ref[...] = jnp.concatenate([i0, i1], axis=1)
    w_ref[...] = jnp.concatenate([w0n, w1n], axis=1)
    r_ref[...] = jnp.concatenate([r0, r1], axis=1).astype(jnp.int32)

    @pl.when(pid == pl.num_programs(0) - 1)
    def _():
        nb = jnp.floor((newc + (BM - 1)) * (1.0 / BM))
        ue = (lax.broadcasted_iota(jnp.int32, (E, E), 0) <=
              lax.broadcasted_iota(jnp.int32, (E, E), 1)).astype(jnp.float32)
        inclt = jnp.dot(nb, ue, preferred_element_type=jnp.float32)
        po_ref[...] = ((inclt - nb) * BM).astype(jnp.int32)
        nbtot = jnp.sum(nb, axis=-1, keepdims=True).astype(jnp.int32)
        erow = lax.broadcasted_iota(jnp.int32, (1, E), 1)
        maxe = jnp.max(jnp.where(nb > 0, erow, 0), axis=-1, keepdims=True)
        tio = lax.broadcasted_iota(jnp.int32, (NBMAX, E), 0).astype(jnp.float32)
        te = jnp.sum(jnp.where(inclt <= tio, 1.0, 0.0), axis=-1,
                     keepdims=True).astype(jnp.int32)
        te_ref[...] = jnp.minimum(te, maxe)
        tcol = lax.broadcasted_iota(jnp.int32, (NBMAX, 1), 0)
        tb_ref[...] = jnp.minimum(tcol, nbtot - 1)
        tv_ref[...] = (tcol < nbtot).astype(jnp.int32)


def _router(scores, ebias_row, *, interpret=False):
    outs = [
        jax.ShapeDtypeStruct((N, K), jnp.int32),
        jax.ShapeDtypeStruct((N, K), jnp.float32),
        jax.ShapeDtypeStruct((N, K), jnp.int32),
        jax.ShapeDtypeStruct((1, E), jnp.int32),
        jax.ShapeDtypeStruct((NBMAX, 1), jnp.int32),
        jax.ShapeDtypeStruct((NBMAX, 1), jnp.int32),
        jax.ShapeDtypeStruct((NBMAX, 1), jnp.int32),
    ]
    tok = lambda i: (i, 0)
    one = lambda i: (0, 0)
    return pl.pallas_call(
        _router_body,
        grid=(N // BT,),
        in_specs=[pl.BlockSpec((BT, E), tok), pl.BlockSpec((1, E), one)],
        out_specs=[pl.BlockSpec((BT, K), tok), pl.BlockSpec((BT, K), tok),
                   pl.BlockSpec((BT, K), tok), pl.BlockSpec((1, E), one),
                   pl.BlockSpec((NBMAX, 1), one), pl.BlockSpec((NBMAX, 1), one),
                   pl.BlockSpec((NBMAX, 1), one)],
        out_shape=outs,
        scratch_shapes=[pltpu.VMEM((1, E), jnp.float32)],
        interpret=interpret,
    )(scores, ebias_row)


NWORK = 32
SLOTS_W = N * K // NWORK
LANES = 16


def _sc_mesh():
    return plsc.VectorSubcoreMesh(core_axis_name="c", subcore_axis_name="s")


def _sc_params():
    cp = pltpu.CompilerParams()
    if "needs_layout_passes" in pltpu.CompilerParams.__dataclass_fields__:
        cp = dataclasses.replace(cp, needs_layout_passes=False)
    return cp


def _sc_route(xf, idxf, rf, pof):
    @functools.partial(
        pl.kernel,
        out_type=(jax.ShapeDtypeStruct((SLOTS, D), jnp.float32),
                  jax.ShapeDtypeStruct((N * K,), jnp.int32)),
        mesh=_sc_mesh(),
        scratch_types=[pltpu.VMEM((E,), jnp.int32),
                       pltpu.VMEM((SLOTS_W,), jnp.int32),
                       pltpu.VMEM((SLOTS_W,), jnp.int32),
                       pltpu.VMEM((SLOTS_W,), jnp.int32),
                       pltpu.VMEM((SLOTS_W,), jnp.int32),
                       pltpu.VMEM((SLOTS_W, D), jnp.float32),
                       pltpu.SemaphoreType.DMA],
        compiler_params=_sc_params(),
    )
    def body(x_hbm, idx_hbm, r_hbm, po_hbm, xs_hbm, pos_hbm,
             po_v, idx_v, r_v, pos_v, dup_v, rows_v, sem):
        wid = lax.axis_index("s") * 2 + lax.axis_index("c")
        base = wid * SLOTS_W
        pltpu.sync_copy(po_hbm, po_v)
        pltpu.sync_copy(idx_hbm.at[pl.ds(base, SLOTS_W)], idx_v)
        pltpu.sync_copy(r_hbm.at[pl.ds(base, SLOTS_W)], r_v)
        for j in range(SLOTS_W // LANES):
            sl = pl.ds(j * LANES, LANES)
            e = idx_v[sl]
            pos_v[sl] = plsc.load_gather(po_v, [e]) + r_v[sl]
            dup_v[sl] = (lax.iota(jnp.int32, LANES) + (base + j * LANES)) >> 1
        pltpu.sync_copy(pos_v, pos_hbm.at[pl.ds(base, SLOTS_W)])
        pltpu.async_copy(x_hbm.at[dup_v], rows_v, sem).wait()
        pltpu.async_copy(rows_v, xs_hbm.at[pos_v], sem).wait()

    return body(xf, idxf, rf, pof)


def _gmm_body(te_ref, tb_ref, tv_ref, xs_ref, gu_ref, dn_ref, y_ref):
    t = pl.program_id(0)

    @pl.when(tv_ref[t] == 1)
    def _():
        h = jnp.dot(xs_ref[...].astype(jnp.bfloat16),
                    gu_ref[0].astype(jnp.bfloat16),
                    preferred_element_type=jnp.float32)
        gate = h[:, :F]
        up = h[:, F:]
        inter = gate * jax.nn.sigmoid(gate) * up
        y_ref[...] = jnp.dot(inter.astype(jnp.bfloat16),
                             dn_ref[0].astype(jnp.bfloat16),
                             preferred_element_type=jnp.float32)


def _gmm(te, tb, tv, xs, gate_up, down, *, interpret=False):
    grid_spec = pltpu.PrefetchScalarGridSpec(
        num_scalar_prefetch=3,
        grid=(NBMAX,),
        in_specs=[
            pl.BlockSpec((BM, D), lambda t, te, tb, tv: (tb[t], 0)),
            pl.BlockSpec((1, D, 2 * F), lambda t, te, tb, tv: (te[t], 0, 0)),
            pl.BlockSpec((1, F, D), lambda t, te, tb, tv: (te[t], 0, 0)),
        ],
        out_specs=pl.BlockSpec((BM, D), lambda t, te, tb, tv: (tb[t], 0)),
    )
    return pl.pallas_call(
        _gmm_body,
        grid_spec=grid_spec,
        out_shape=jax.ShapeDtypeStruct((SLOTS, D), jnp.float32),
        interpret=interpret,
    )(te, tb, tv, xs, gate_up, down)


def _sc_unroute(y, pos):
    @functools.partial(
        pl.kernel,
        out_type=jax.ShapeDtypeStruct((N * K, D), jnp.float32),
        mesh=_sc_mesh(),
        scratch_types=[pltpu.VMEM((SLOTS_W,), jnp.int32),
                       pltpu.VMEM((SLOTS_W, D), jnp.float32),
                       pltpu.SemaphoreType.DMA],
    )
    def body(y_hbm, pos_hbm, g_hbm, pos_v, rows_v, sem):
        wid = lax.axis_index("s") * 2 + lax.axis_index("c")
        base = wid * SLOTS_W
        pltpu.sync_copy(pos_hbm.at[pl.ds(base, SLOTS_W)], pos_v)
        pltpu.async_copy(y_hbm.at[pos_v], rows_v, sem).wait()
        pltpu.sync_copy(rows_v, g_hbm.at[pl.ds(base, SLOTS_W)])

    return body(y, pos)


def _combine_body(x_ref, sgu_ref, sdn_ref, g_ref, w_ref, o_ref):
    h = jnp.dot(x_ref[...].astype(jnp.bfloat16),
                sgu_ref[...].astype(jnp.bfloat16),
                preferred_element_type=jnp.float32)
    gate = h[:, :SF]
    up = h[:, SF:]
    inter = gate * jax.nn.sigmoid(gate) * up
    shared = jnp.dot(inter.astype(jnp.bfloat16),
                     sdn_ref[...].astype(jnp.bfloat16),
                     preferred_element_type=jnp.float32)
    w = w_ref[...]
    o_ref[...] = (shared + w[:, 0:1] * g_ref[:, :D]
                  + w[:, 1:2] * g_ref[:, D:])


def _combine(xf, s_gate_up, s_down, g2, w, *, interpret=False):
    tok = lambda i: (i, 0)
    one = lambda i: (0, 0)
    return pl.pallas_call(
        _combine_body,
        grid=(N // BT,),
        in_specs=[pl.BlockSpec((BT, D), tok),
                  pl.BlockSpec((D, 2 * SF), one),
                  pl.BlockSpec((SF, D), one),
                  pl.BlockSpec((BT, 2 * D), tok),
                  pl.BlockSpec((BT, K), tok)],
        out_specs=pl.BlockSpec((BT, D), tok),
        out_shape=jax.ShapeDtypeStruct((N, D), jnp.float32),
        interpret=interpret,
    )(xf, s_gate_up, s_down, g2, w)


def kernel(x, gate_w, e_bias, gate_up, down, s_gate_up, s_down):
    xf = x.reshape(N, D)
    logits = xf @ gate_w
    scores = jax.nn.sigmoid(logits.astype(jnp.float32))
    idx, w, r, po, te, tb, tv = _router(scores, e_bias.reshape(1, E))
    xs, pos = _sc_route(xf, idx.reshape(-1), r.reshape(-1), po.reshape(-1))
    y = _gmm(te.reshape(-1), tb.reshape(-1), tv.reshape(-1),
             xs, gate_up, down)
    g = _sc_unroute(y, pos)
    out = _combine(xf, s_gate_up, s_down, g.reshape(N, 2 * D), w)
    return out.reshape(1, N, D), idx, scores


# speedup vs baseline: 1.0085x; 1.0085x over previous
import dataclasses
import functools

import jax
import jax.numpy as jnp
from jax import lax
from jax.experimental import pallas as pl
from jax.experimental.pallas import tpu as pltpu
from jax.experimental.pallas import tpu_sc as plsc

N = 2048
D = 768
E = 64
K = 2
G = 8
EPG = E // G
TG = 4
F = 512
SF = 512
SCALE = 2.5

BT = 256
BM = 128
NBMAX = E + N * K // BM - 32
SLOTS = NBMAX * BM

_NEG = -1e30


def _router_body(scores_ref, ebias_ref, idx_ref, w_ref, r_ref, po_ref,
                 te_ref, tb_ref, tv_ref, carry_ref):
    pid = pl.program_id(0)

    @pl.when(pid == 0)
    def _():
        carry_ref[...] = jnp.zeros((1, E), jnp.float32)

    s = scores_ref[...]
    sr = s + ebias_ref[...]
    eid = lax.broadcasted_iota(jnp.int32, (BT, E), 1)
    gid = eid >> 3

    def _bfly(v, op):
        for sh in (1, 2, 4):
            up = pltpu.roll(v, E - sh, axis=1)
            dn = pltpu.roll(v, sh, axis=1)
            v = op(v, jnp.where((eid & sh) != 0, dn, up))
        return v

    m1 = _bfly(sr, jnp.maximum)
    i1 = _bfly(jnp.where(sr == m1, eid, E), jnp.minimum)
    m2 = _bfly(jnp.where(eid == i1, _NEG, sr), jnp.maximum)
    gsum = m1 + m2

    sel = jnp.zeros((BT, E), jnp.bool_)
    gm = gsum
    for _ in range(TG):
        mg = jnp.max(gm, axis=-1, keepdims=True)
        lg = jnp.min(jnp.where(gm == mg, eid, E), axis=-1, keepdims=True)
        hit = gid == (lg >> 3)
        sel = sel | hit
        gm = jnp.where(hit, _NEG, gm)

    msk = jnp.where(sel, sr, _NEG)
    picks = []
    for _ in range(K):
        mk = jnp.max(msk, axis=-1, keepdims=True)
        ik = jnp.min(jnp.where(msk == mk, eid, E), axis=-1, keepdims=True)
        wk = jnp.sum(jnp.where(eid == ik, s, 0.0), axis=-1, keepdims=True)
        msk = jnp.where(eid == ik, _NEG, msk)
        picks.append((ik, wk))
    (i0, w0), (i1, w1) = picks
    denom = (w0 + w1) + 1e-20
    w0n = (w0 / denom) * SCALE
    w1n = (w1 / denom) * SCALE

    onehots = (eid == i0).astype(jnp.float32) + (eid == i1).astype(jnp.float32)
    rio = lax.broadcasted_iota(jnp.int32, (BT, E), 0)
    cs = onehots
    sh = 1
    while sh < BT:
        cs = cs + jnp.where(rio >= sh, pltpu.roll(cs, sh, axis=0), 0.0)
        sh *= 2
    tot = cs + carry_ref[...]
    r0 = jnp.sum(jnp.where(eid == i0, tot, 0.0), axis=-1, keepdims=True) - 1.0
    r1 = jnp.sum(jnp.where(eid == i1, tot, 0.0), axis=-1, keepdims=True) - 1.0
    newc = carry_ref[...] + cs[BT - 1:BT, :]
    carry_ref[...] = newc

    idx_ref[...] = jnp.concatenate([i0, i1], axis=1)
    w_ref[...] = jnp.concatenate([w0n, w1n], axis=1)
    r_ref[...] = jnp.concatenate([r0, r1], axis=1).astype(jnp.int32)

    @pl.when(pid == pl.num_programs(0) - 1)
    def _():
        nb = jnp.floor((newc + (BM - 1)) * (1.0 / BM))
        ue = (lax.broadcasted_iota(jnp.int32, (E, E), 0) <=
              lax.broadcasted_iota(jnp.int32, (E, E), 1)).astype(jnp.float32)
        inclt = jnp.dot(nb, ue, preferred_element_type=jnp.float32)
        po_ref[...] = ((inclt - nb) * BM).astype(jnp.int32)
        nbtot = jnp.sum(nb, axis=-1, keepdims=True).astype(jnp.int32)
        erow = lax.broadcasted_iota(jnp.int32, (1, E), 1)
        maxe = jnp.max(jnp.where(nb > 0, erow, 0), axis=-1, keepdims=True)
        tio = lax.broadcasted_iota(jnp.int32, (NBMAX, E), 0).astype(jnp.float32)
        te = jnp.sum(jnp.where(inclt <= tio, 1.0, 0.0), axis=-1,
                     keepdims=True).astype(jnp.int32)
        te_ref[...] = jnp.minimum(te, maxe)
        tcol = lax.broadcasted_iota(jnp.int32, (NBMAX, 1), 0)
        tb_ref[...] = jnp.minimum(tcol, nbtot - 1)
        tv_ref[...] = (tcol < nbtot).astype(jnp.int32)


def _router(scores, ebias_row, *, interpret=False):
    outs = [
        jax.ShapeDtypeStruct((N, K), jnp.int32),
        jax.ShapeDtypeStruct((N, K), jnp.float32),
        jax.ShapeDtypeStruct((N, K), jnp.int32),
        jax.ShapeDtypeStruct((1, E), jnp.int32),
        jax.ShapeDtypeStruct((NBMAX, 1), jnp.int32),
        jax.ShapeDtypeStruct((NBMAX, 1), jnp.int32),
        jax.ShapeDtypeStruct((NBMAX, 1), jnp.int32),
    ]
    tok = lambda i: (i, 0)
    one = lambda i: (0, 0)
    return pl.pallas_call(
        _router_body,
        grid=(N // BT,),
        in_specs=[pl.BlockSpec((BT, E), tok), pl.BlockSpec((1, E), one)],
        out_specs=[pl.BlockSpec((BT, K), tok), pl.BlockSpec((BT, K), tok),
                   pl.BlockSpec((BT, K), tok), pl.BlockSpec((1, E), one),
                   pl.BlockSpec((NBMAX, 1), one), pl.BlockSpec((NBMAX, 1), one),
                   pl.BlockSpec((NBMAX, 1), one)],
        out_shape=outs,
        scratch_shapes=[pltpu.VMEM((1, E), jnp.float32)],
        interpret=interpret,
    )(scores, ebias_row)


NWORK = 32
SLOTS_W = N * K // NWORK
LANES = 16


def _sc_mesh():
    return plsc.VectorSubcoreMesh(core_axis_name="c", subcore_axis_name="s")


def _sc_params():
    cp = pltpu.CompilerParams()
    if "needs_layout_passes" in pltpu.CompilerParams.__dataclass_fields__:
        cp = dataclasses.replace(cp, needs_layout_passes=False)
    return cp


def _sc_route(xf, idxf, rf, pof):
    @functools.partial(
        pl.kernel,
        out_type=(jax.ShapeDtypeStruct((SLOTS, D), jnp.float32),
                  jax.ShapeDtypeStruct((N * K,), jnp.int32)),
        mesh=_sc_mesh(),
        scratch_types=[pltpu.VMEM((E,), jnp.int32),
                       pltpu.VMEM((SLOTS_W,), jnp.int32),
                       pltpu.VMEM((SLOTS_W,), jnp.int32),
                       pltpu.VMEM((SLOTS_W,), jnp.int32),
                       pltpu.VMEM((SLOTS_W,), jnp.int32),
                       pltpu.VMEM((SLOTS_W, D), jnp.float32),
                       pltpu.SemaphoreType.DMA],
        compiler_params=_sc_params(),
    )
    def body(x_hbm, idx_hbm, r_hbm, po_hbm, xs_hbm, pos_hbm,
             po_v, idx_v, r_v, pos_v, dup_v, rows_v, sem):
        wid = lax.axis_index("s") * 2 + lax.axis_index("c")
        base = wid * SLOTS_W
        pltpu.sync_copy(po_hbm, po_v)
        pltpu.sync_copy(idx_hbm.at[pl.ds(base, SLOTS_W)], idx_v)
        pltpu.sync_copy(r_hbm.at[pl.ds(base, SLOTS_W)], r_v)
        for j in range(SLOTS_W // LANES):
            sl = pl.ds(j * LANES, LANES)
            e = idx_v[sl]
            pos_v[sl] = plsc.load_gather(po_v, [e]) + r_v[sl]
            dup_v[sl] = (lax.iota(jnp.int32, LANES) + (base + j * LANES)) >> 1
        pltpu.sync_copy(pos_v, pos_hbm.at[pl.ds(base, SLOTS_W)])
        pltpu.async_copy(x_hbm.at[dup_v], rows_v, sem).wait()
        pltpu.async_copy(rows_v, xs_hbm.at[pos_v], sem).wait()

    return body(xf, idxf, rf, pof)


def _gmm_body(te_ref, tb_ref, tv_ref, xs_ref, gu_ref, dn_ref, y_ref):
    t = pl.program_id(0)

    @pl.when(tv_ref[t] == 1)
    def _():
        h = jnp.dot(xs_ref[...].astype(jnp.bfloat16),
                    gu_ref[0].astype(jnp.bfloat16),
                    preferred_element_type=jnp.float32)
        gate = h[:, :F]
        up = h[:, F:]
        inter = gate * jax.nn.sigmoid(gate) * up
        y_ref[...] = jnp.dot(inter.astype(jnp.bfloat16),
                             dn_ref[0].astype(jnp.bfloat16),
                             preferred_element_type=jnp.float32
                             ).astype(jnp.bfloat16)


def _gmm(te, tb, tv, xs, gate_up, down, *, interpret=False):
    grid_spec = pltpu.PrefetchScalarGridSpec(
        num_scalar_prefetch=3,
        grid=(NBMAX,),
        in_specs=[
            pl.BlockSpec((BM, D), lambda t, te, tb, tv: (tb[t], 0)),
            pl.BlockSpec((1, D, 2 * F), lambda t, te, tb, tv: (te[t], 0, 0)),
            pl.BlockSpec((1, F, D), lambda t, te, tb, tv: (te[t], 0, 0)),
        ],
        out_specs=pl.BlockSpec((BM, D), lambda t, te, tb, tv: (tb[t], 0)),
    )
    return pl.pallas_call(
        _gmm_body,
        grid_spec=grid_spec,
        out_shape=jax.ShapeDtypeStruct((SLOTS, D), jnp.bfloat16),
        interpret=interpret,
    )(te, tb, tv, xs, gate_up, down)


def _sc_unroute(y, pos):
    @functools.partial(
        pl.kernel,
        out_type=jax.ShapeDtypeStruct((N * K, D), jnp.bfloat16),
        mesh=_sc_mesh(),
        scratch_types=[pltpu.VMEM((SLOTS_W,), jnp.int32),
                       pltpu.VMEM((SLOTS_W, D), jnp.bfloat16),
                       pltpu.SemaphoreType.DMA],
    )
    def body(y_hbm, pos_hbm, g_hbm, pos_v, rows_v, sem):
        wid = lax.axis_index("s") * 2 + lax.axis_index("c")
        base = wid * SLOTS_W
        pltpu.sync_copy(pos_hbm.at[pl.ds(base, SLOTS_W)], pos_v)
        pltpu.async_copy(y_hbm.at[pos_v], rows_v, sem).wait()
        pltpu.sync_copy(rows_v, g_hbm.at[pl.ds(base, SLOTS_W)])

    return body(y, pos)


def _combine_body(x_ref, sgu_ref, sdn_ref, g_ref, w_ref, o_ref):
    h = jnp.dot(x_ref[...].astype(jnp.bfloat16),
                sgu_ref[...].astype(jnp.bfloat16),
                preferred_element_type=jnp.float32)
    gate = h[:, :SF]
    up = h[:, SF:]
    inter = gate * jax.nn.sigmoid(gate) * up
    shared = jnp.dot(inter.astype(jnp.bfloat16),
                     sdn_ref[...].astype(jnp.bfloat16),
                     preferred_element_type=jnp.float32)
    w = w_ref[...]
    g = g_ref[...].astype(jnp.float32)
    o_ref[...] = shared + w[:, 0:1] * g[:, :D] + w[:, 1:2] * g[:, D:]


def _combine(xf, s_gate_up, s_down, g2, w, *, interpret=False):
    tok = lambda i: (i, 0)
    one = lambda i: (0, 0)
    return pl.pallas_call(
        _combine_body,
        grid=(N // BT,),
        in_specs=[pl.BlockSpec((BT, D), tok),
                  pl.BlockSpec((D, 2 * SF), one),
                  pl.BlockSpec((SF, D), one),
                  pl.BlockSpec((BT, 2 * D), tok),
                  pl.BlockSpec((BT, K), tok)],
        out_specs=pl.BlockSpec((BT, D), tok),
        out_shape=jax.ShapeDtypeStruct((N, D), jnp.float32),
        interpret=interpret,
    )(xf, s_gate_up, s_down, g2, w)


def kernel(x, gate_w, e_bias, gate_up, down, s_gate_up, s_down):
    xf = x.reshape(N, D)
    logits = xf @ gate_w
    scores = jax.nn.sigmoid(logits.astype(jnp.float32))
    idx, w, r, po, te, tb, tv = _router(scores, e_bias.reshape(1, E))
    xs, pos = _sc_route(xf, idx.reshape(-1), r.reshape(-1), po.reshape(-1))
    y = _gmm(te.reshape(-1), tb.reshape(-1), tv.reshape(-1),
             xs, gate_up, down)
    g = _sc_unroute(y, pos)
    out = _combine(xf, s_gate_up, s_down, g.reshape(N, 2 * D), w)
    return out.reshape(1, N, D), idx, scores
